# baseline (speedup 1.0000x reference)
import functools
import math

import numpy as np
import jax
import jax.numpy as jnp
from jax import lax
from jax.experimental import pallas as pl
from jax.experimental.pallas import tpu as pltpu
from jax.experimental.pallas import tpu_sc as plsc

D_MODEL = 768
MAX_SEQ_LEN = 2048
_SCALE = math.sqrt(float(D_MODEL))
_LANES = 16


def _pe_host() -> np.ndarray:
    pos = np.arange(MAX_SEQ_LEN, dtype=np.float64).reshape(-1, 1)
    i = np.arange(D_MODEL, dtype=np.float64)
    rads = pos / np.power(10000.0, 2.0 * np.floor(i / 2.0) / D_MODEL)
    pe = np.zeros((MAX_SEQ_LEN, D_MODEL), dtype=np.float32)
    pe[:, 0::2] = np.sin(rads[:, 0::2]).astype(np.float32)
    pe[:, 1::2] = np.cos(rads[:, 1::2]).astype(np.float32)
    return pe


_PE = _pe_host()


@functools.lru_cache(maxsize=None)
def _build(batch: int, seq: int):
    info = plsc.get_sparse_core_info()
    nc, ns = info.num_cores, info.num_subcores
    nw = nc * ns
    rpw = batch * seq // nw
    wpb = nw // batch
    chunk = 16
    nchunk = rpw // chunk
    groups = D_MODEL // _LANES

    mesh = plsc.VectorSubcoreMesh(core_axis_name="c", subcore_axis_name="s")

    @functools.partial(
        pl.kernel,
        mesh=mesh,
        out_type=jax.ShapeDtypeStruct((batch, seq, D_MODEL), jnp.float32),
        scratch_types=[
            pltpu.VMEM((rpw,), jnp.int32),
            pltpu.VMEM((2, chunk, D_MODEL), jnp.float32),
            pltpu.VMEM((2, chunk, D_MODEL), jnp.float32),
            pltpu.SemaphoreType.DMA,
            pltpu.SemaphoreType.DMA,
            pltpu.SemaphoreType.DMA,
            pltpu.SemaphoreType.DMA,
            pltpu.SemaphoreType.DMA,
            pltpu.SemaphoreType.DMA,
        ],
    )
    def emb(x_hbm, table_hbm, pe_hbm, out_hbm, idx_v, tok_v, pe_v,
            sg0, sg1, sp0, sp1, so0, so1):
        sg, sp, so = (sg0, sg1), (sp0, sp1), (so0, so1)
        wid = lax.axis_index("s") * nc + lax.axis_index("c")
        bi = wid // wpb
        seq0 = (wid % wpb) * rpw
        pltpu.sync_copy(x_hbm.at[bi, pl.ds(seq0, rpw)], idx_v)

        def start_in(c):
            b = c & 1
            return (
                pltpu.async_copy(
                    table_hbm.at[idx_v.at[pl.ds(c * chunk, chunk)]],
                    tok_v.at[b], sg[b]),
                pltpu.async_copy(
                    pe_hbm.at[pl.ds(seq0 + c * chunk, chunk)],
                    pe_v.at[b], sp[b]),
            )

        pend_in = {0: start_in(0)}
        pend_out = {}
        for c in range(nchunk):
            b = c & 1
            if c + 1 < nchunk:
                if c - 1 in pend_out:
                    pend_out.pop(c - 1).wait()
                pend_in[c + 1] = start_in(c + 1)
            g, p = pend_in.pop(c)
            g.wait()
            p.wait()

            @plsc.parallel_loop(0, chunk, unroll=2)
            def _row(r, b=b):
                for gi in range(groups):
                    sl = pl.ds(gi * _LANES, _LANES)
                    tok_v[b, r, sl] = tok_v[b, r, sl] * _SCALE + pe_v[b, r, sl]

            pend_out[c] = pltpu.async_copy(
                tok_v.at[b], out_hbm.at[bi, pl.ds(seq0 + c * chunk, chunk)],
                so[b])
        for c in sorted(pend_out):
            pend_out.pop(c).wait()

    return emb


def kernel(x, table):
    b, s = x.shape
    emb = _build(b, s)
    pe = jnp.asarray(_PE)
    return emb(x, table, pe)


# speedup vs baseline: 1.0339x; 1.0339x over previous
import functools
import math

import numpy as np
import jax
import jax.numpy as jnp
from jax import lax
from jax.experimental import pallas as pl
from jax.experimental.pallas import tpu as pltpu
from jax.experimental.pallas import tpu_sc as plsc

D_MODEL = 768
MAX_SEQ_LEN = 2048
_SCALE = math.sqrt(float(D_MODEL))
_LANES = 16


def _pe_host() -> np.ndarray:
    pos = np.arange(MAX_SEQ_LEN, dtype=np.float64).reshape(-1, 1)
    i = np.arange(D_MODEL, dtype=np.float64)
    rads = pos / np.power(10000.0, 2.0 * np.floor(i / 2.0) / D_MODEL)
    pe = np.zeros((MAX_SEQ_LEN, D_MODEL), dtype=np.float32)
    pe[:, 0::2] = np.sin(rads[:, 0::2]).astype(np.float32)
    pe[:, 1::2] = np.cos(rads[:, 1::2]).astype(np.float32)
    return pe


_PE = _pe_host()


@functools.lru_cache(maxsize=None)
def _build(batch: int, seq: int):
    info = plsc.get_sparse_core_info()
    nc, ns = info.num_cores, info.num_subcores
    nw = nc * ns
    rpw = batch * seq // nw
    wpb = nw // batch
    chunk = 32
    nchunk = rpw // chunk
    groups = D_MODEL // _LANES
    ntok = 3
    npe = 2

    mesh = plsc.VectorSubcoreMesh(core_axis_name="c", subcore_axis_name="s")

    @functools.partial(
        pl.kernel,
        mesh=mesh,
        out_type=jax.ShapeDtypeStruct((batch, seq, D_MODEL), jnp.float32),
        scratch_types=[
            pltpu.VMEM((rpw,), jnp.int32),
            pltpu.VMEM((ntok, chunk, D_MODEL), jnp.float32),
            pltpu.VMEM((npe, chunk, D_MODEL), jnp.float32),
            pltpu.SemaphoreType.DMA,
            pltpu.SemaphoreType.DMA,
            pltpu.SemaphoreType.DMA,
            pltpu.SemaphoreType.DMA,
            pltpu.SemaphoreType.DMA,
            pltpu.SemaphoreType.DMA,
            pltpu.SemaphoreType.DMA,
        ],
    )
    def emb(x_hbm, table_hbm, pe_hbm, out_hbm, idx_v, tok_v, pe_v,
            sg0, sg1, sg2, sp0, sp1, so0, so1):
        sg, sp, so = (sg0, sg1, sg2), (sp0, sp1), (so0, so1)
        wid = lax.axis_index("s") * nc + lax.axis_index("c")
        bi = wid // wpb
        seq0 = (wid % wpb) * rpw
        pltpu.sync_copy(x_hbm.at[bi, pl.ds(seq0, rpw)], idx_v)

        def start_g(c):
            t = c % ntok
            return pltpu.async_copy(
                table_hbm.at[idx_v.at[pl.ds(c * chunk, chunk)]],
                tok_v.at[t], sg[t])

        def start_p(c):
            q = c % npe
            return pltpu.async_copy(
                pe_hbm.at[pl.ds(seq0 + c * chunk, chunk)], pe_v.at[q], sp[q])

        pend_g = {0: start_g(0), 1: start_g(1)}
        pend_p = {0: start_p(0)}
        pend_o = {}
        for c in range(nchunk):
            t = c % ntok
            q = c % npe
            if c + 2 < nchunk:
                if c - 1 in pend_o:
                    pend_o.pop(c - 1).wait()
                pend_g[c + 2] = start_g(c + 2)
            if c + 1 < nchunk:
                pend_p[c + 1] = start_p(c + 1)
            pend_g.pop(c).wait()
            pend_p.pop(c).wait()

            @plsc.parallel_loop(0, chunk, unroll=2)
            def _row(r, t=t, q=q):
                for gi in range(groups):
                    sl = pl.ds(gi * _LANES, _LANES)
                    tok_v[t, r, sl] = tok_v[t, r, sl] * _SCALE + pe_v[q, r, sl]

            pend_o[c] = pltpu.async_copy(
                tok_v.at[t], out_hbm.at[bi, pl.ds(seq0 + c * chunk, chunk)],
                so[c & 1])
        for c in sorted(pend_o):
            pend_o.pop(c).wait()

    return emb


def kernel(x, table):
    b, s = x.shape
    emb = _build(b, s)
    pe = jnp.asarray(_PE)
    return emb(x, table, pe)


# speedup vs baseline: 1.0842x; 1.0487x over previous
import functools
import math

import numpy as np
import jax
import jax.numpy as jnp
from jax import lax
from jax.experimental import pallas as pl
from jax.experimental.pallas import tpu as pltpu
from jax.experimental.pallas import tpu_sc as plsc

D_MODEL = 768
MAX_SEQ_LEN = 2048
_SCALE = math.sqrt(float(D_MODEL))
_LANES = 16


def _pe_host() -> np.ndarray:
    pos = np.arange(MAX_SEQ_LEN, dtype=np.float64).reshape(-1, 1)
    i = np.arange(D_MODEL, dtype=np.float64)
    rads = pos / np.power(10000.0, 2.0 * np.floor(i / 2.0) / D_MODEL)
    pe = np.zeros((MAX_SEQ_LEN, D_MODEL), dtype=np.float32)
    pe[:, 0::2] = np.sin(rads[:, 0::2]).astype(np.float32)
    pe[:, 1::2] = np.cos(rads[:, 1::2]).astype(np.float32)
    return pe


_PE = _pe_host()


@functools.lru_cache(maxsize=None)
def _build(batch: int, seq: int):
    info = plsc.get_sparse_core_info()
    nc, ns = info.num_cores, info.num_subcores
    nw = nc * ns
    rpw = batch * seq // nw
    wpb = nw // batch
    chunk = 32
    nchunk = rpw // chunk
    groups = D_MODEL // _LANES

    mesh = plsc.VectorSubcoreMesh(core_axis_name="c", subcore_axis_name="s")

    @functools.partial(
        pl.kernel,
        mesh=mesh,
        out_type=jax.ShapeDtypeStruct((batch, seq, D_MODEL), jnp.float32),
        scratch_types=[
            pltpu.VMEM((rpw,), jnp.int32),
            pltpu.VMEM((2, chunk, D_MODEL), jnp.float32),
            pltpu.VMEM((2, chunk, D_MODEL), jnp.float32),
            pltpu.SemaphoreType.DMA,
            pltpu.SemaphoreType.DMA,
            pltpu.SemaphoreType.DMA,
            pltpu.SemaphoreType.DMA,
            pltpu.SemaphoreType.DMA,
            pltpu.SemaphoreType.DMA,
        ],
    )
    def emb(x_hbm, table_hbm, pe_hbm, out_hbm, idx_v, tok_v, pe_v,
            sg0, sg1, sp0, sp1, so0, so1):
        sg, sp, so = (sg0, sg1), (sp0, sp1), (so0, so1)
        wid = lax.axis_index("s") * nc + lax.axis_index("c")
        bi = wid // wpb
        seq0 = (wid % wpb) * rpw
        pltpu.sync_copy(x_hbm.at[bi, pl.ds(seq0, rpw)], idx_v)

        def start_g(c):
            b = c & 1
            return pltpu.async_copy(
                table_hbm.at[idx_v.at[pl.ds(c * chunk, chunk)]],
                tok_v.at[b], sg[b])

        def start_p(c):
            b = c & 1
            return pltpu.async_copy(
                pe_hbm.at[pl.ds(seq0 + c * chunk, chunk)], pe_v.at[b], sp[b])

        pend_g = {0: start_g(0)}
        pend_p = {0: start_p(0)}
        pend_o = {}
        for c in range(nchunk):
            b = c & 1
            if c + 1 < nchunk:
                pend_p[c + 1] = start_p(c + 1)
                if c - 1 in pend_o:
                    pend_o.pop(c - 1).wait()
                pend_g[c + 1] = start_g(c + 1)
            pend_g.pop(c).wait()
            pend_p.pop(c).wait()

            @plsc.parallel_loop(0, chunk, unroll=2)
            def _row(r, b=b):
                for gi in range(groups):
                    sl = pl.ds(gi * _LANES, _LANES)
                    tok_v[b, r, sl] = tok_v[b, r, sl] * _SCALE + pe_v[b, r, sl]

            pend_o[c] = pltpu.async_copy(
                tok_v.at[b], out_hbm.at[bi, pl.ds(seq0 + c * chunk, chunk)],
                so[b])
        for c in sorted(pend_o):
            pend_o.pop(c).wait()

    return emb


def kernel(x, table):
    b, s = x.shape
    emb = _build(b, s)
    pe = jnp.asarray(_PE)
    return emb(x, table, pe)
